# baseline (speedup 1.0000x reference)
import numpy as np
import jax
import jax.numpy as jnp
from jax.experimental import pallas as pl
from jax.experimental.pallas import tpu as pltpu

_B = 8
_C = 8
_OC = 16
_KH = 3
_KW = 3
_K = 2
_H = 28
_W = 28
_L = _H * _W
_N = _B * _L
_R = _C * _KH * _KW
_NPAIR = _C * 36

_PAIRS = [(u, v) for u in range(9) for v in range(u + 1, 9)]


def _build_mask_rows():
    rng = np.random.RandomState(0)
    rows = []
    for _o in range(_OC):
        for ci in range(_C):
            sels = [(ci, kh, kw) for kh in range(_KH) for kw in range(_KW)]
            for kh in range(_KH):
                for kw in range(_KW):
                    conv = (ci, kh, kw)
                    sub = [s for s in sels if s != conv]
                    rows.append(conv)
                    for _ in range(_K - 1):
                        rows.append(sub[rng.randint(len(sub))])
    return np.asarray(rows, dtype=np.int64)


def _build_constants():
    mask_rows = _build_mask_rows()

    t4 = np.zeros((4, 4), np.float64)
    for c in range(4):
        s0 = 1.0 if (c & 1) else -1.0
        s1 = 1.0 if (c & 2) else -1.0
        t4[c, 0] = 0.25
        t4[c, 1] = 0.25 * s0
        t4[c, 2] = 0.25 * s1
        t4[c, 3] = 0.25 * s0 * s1

    pairid = {p: i for i, p in enumerate(_PAIRS)}

    kw01p = np.zeros((288, _R), np.float64)
    kw10 = np.zeros((288, _R), np.float64)
    kw11 = np.zeros((288, _R), np.float64)
    kc = np.zeros((288, 1), np.float64)
    pb10 = np.zeros((_OC * _R, _R), np.float32)
    pb11 = np.zeros((_OC * _R, _NPAIR), np.float32)
    for o2 in range(_OC):
        for r in range(_R):
            t = o2 * _R + r
            ci, p0 = divmod(r, 9)
            ci1, kh1, kw1 = mask_rows[2 * t + 1]
            p1 = int(kh1) * 3 + int(kw1)
            j0 = p0 * 8 + ci
            q = pairid[(min(p0, p1), max(p0, p1))] * 8 + int(ci1)
            j1 = p1 * 8 + int(ci1)
            for c in range(4):
                m = r * 4 + c
                if o2 == 0:
                    kw01p[m, j0] += t4[c, 1]
                    kc[m, 0] += t4[c, 0]
                kw10[m, r] = t4[c, 2]
                kw11[m, r] = t4[c, 3]
            pb10[t, j1] = 1.0
            pb11[t, q] = 1.0

    shifts = [(kh - 1) * _W + (kw - 1) for kh in range(_KH) for kw in range(_KW)]
    kwall = np.zeros((288, 360), np.float64)
    kwall[:, 0:72] = kw01p
    kwall[:, 72:144] = kw10
    kwall[:, 144:216] = kw11
    kwall[:, 216:217] = kc
    pball = np.concatenate([pb10, pb11], axis=1)
    return (kwall.astype(np.float32), pball, shifts)


(_KWALL, _PBALL, _SHIFTS) = _build_constants()


def _body(x_ref, tt_ref, kwall_hbm, pball_hbm, out_ref,
          kwall_v, pball_v, sem1, sem2):
    f32 = jnp.float32
    bf16 = jnp.bfloat16

    cp1 = pltpu.make_async_copy(kwall_hbm, kwall_v, sem1)
    cp2 = pltpu.make_async_copy(pball_hbm, pball_v, sem2)
    cp1.start()
    cp2.start()

    x = x_ref[...]
    s = jnp.sign(x)
    pos = jax.lax.broadcasted_iota(jnp.int32, (1, _L), 1)
    oh = pos // _W
    ow = pos % _W
    parts = []
    for kh in range(_KH):
        for kw in range(_KW):
            d = ((kh - 1) * _W + (kw - 1)) % _L
            if d == 0:
                rolled = s
            else:
                rolled = jnp.concatenate([s[:, d:], s[:, :d]], axis=1)
            ok = (oh + kh - 1 >= 0) & (oh + kh - 1 < _H) & \
                 (ow + kw - 1 >= 0) & (ow + kw - 1 < _W)
            parts.append(rolled * ok.astype(f32))

    rhs_all = []
    for b in range(_B):
        lo = b * _C
        sb = [p[lo:lo + _C] for p in parts]
        pb = [sb[u] * sb[v] for (u, v) in _PAIRS]
        rhs_all.append(jnp.concatenate(sb + pb, axis=0).astype(bf16))

    cp1.wait()
    cp2.wait()
    ttb = tt_ref[...]
    small = jnp.dot(ttb, kwall_v[...], preferred_element_type=f32)
    lmat1 = small[:, 0:72]
    w10 = small[:, 72:144]
    w11 = small[:, 144:216]
    cvec = small[:, 216:217]
    col = jax.lax.broadcasted_iota(jnp.int32, (_OC, _OC * _R), 1) // _R
    row = jax.lax.broadcasted_iota(jnp.int32, (_OC, _OC * _R), 0)
    maskd = (col == row).astype(f32)
    d1011 = jnp.concatenate(
        [jnp.concatenate([w10] * _OC, axis=1) * maskd,
         jnp.concatenate([w11] * _OC, axis=1) * maskd], axis=0)
    lm2mm = jnp.dot(d1011, pball_v[...], preferred_element_type=f32)
    lmat = lmat1 + lm2mm[0:_OC, 0:72]
    mmat = lm2mm[_OC:2 * _OC, 72:360]
    lm = jnp.concatenate([lmat, mmat], axis=1).astype(bf16)

    for b in range(_B):
        out_ref[b] = jnp.dot(lm, rhs_all[b], preferred_element_type=f32) + cvec


def kernel(input, truth_tables):
    x = input.reshape(_B * _C, _L)
    ttb = truth_tables.reshape(_OC, _R * 4)
    out = pl.pallas_call(
        _body,
        out_shape=jax.ShapeDtypeStruct((_B, _OC, _L), jnp.float32),
        in_specs=[
            pl.BlockSpec(memory_space=pltpu.VMEM),
            pl.BlockSpec(memory_space=pltpu.VMEM),
            pl.BlockSpec(memory_space=pl.ANY),
            pl.BlockSpec(memory_space=pl.ANY),
        ],
        scratch_shapes=[
            pltpu.VMEM((288, 360), jnp.float32),
            pltpu.VMEM((_OC * _R, 360), jnp.float32),
            pltpu.SemaphoreType.DMA,
            pltpu.SemaphoreType.DMA,
        ],
    )(x, ttb, _KWALL, _PBALL)
    return out.reshape(_B, _OC, _H, _W)


# speedup vs baseline: 1.0669x; 1.0669x over previous
import numpy as np
import jax
import jax.numpy as jnp
from jax.experimental import pallas as pl
from jax.experimental.pallas import tpu as pltpu

_B = 8
_C = 8
_OC = 16
_KH = 3
_KW = 3
_K = 2
_H = 28
_W = 28
_L = _H * _W
_N = _B * _L
_R = _C * _KH * _KW
_NPAIR = _C * 36

_PAIRS = [(u, v) for u in range(9) for v in range(u + 1, 9)]


def _build_mask_rows():
    rng = np.random.RandomState(0)
    rows = []
    for _o in range(_OC):
        for ci in range(_C):
            sels = [(ci, kh, kw) for kh in range(_KH) for kw in range(_KW)]
            for kh in range(_KH):
                for kw in range(_KW):
                    conv = (ci, kh, kw)
                    sub = [s for s in sels if s != conv]
                    rows.append(conv)
                    for _ in range(_K - 1):
                        rows.append(sub[rng.randint(len(sub))])
    return np.asarray(rows, dtype=np.int64)


def _build_constants():
    mask_rows = _build_mask_rows()

    t4 = np.zeros((4, 4), np.float64)
    for c in range(4):
        s0 = 1.0 if (c & 1) else -1.0
        s1 = 1.0 if (c & 2) else -1.0
        t4[c, 0] = 0.25
        t4[c, 1] = 0.25 * s0
        t4[c, 2] = 0.25 * s1
        t4[c, 3] = 0.25 * s0 * s1

    pairid = {p: i for i, p in enumerate(_PAIRS)}

    kw01p = np.zeros((288, _R), np.float64)
    kw10 = np.zeros((288, _R), np.float64)
    kw11 = np.zeros((288, _R), np.float64)
    kc = np.zeros((288, 1), np.float64)
    pb10 = np.zeros((_OC * _R, _R), np.float32)
    pb11 = np.zeros((_OC * _R, _NPAIR), np.float32)
    for o2 in range(_OC):
        for r in range(_R):
            t = o2 * _R + r
            ci, p0 = divmod(r, 9)
            ci1, kh1, kw1 = mask_rows[2 * t + 1]
            p1 = int(kh1) * 3 + int(kw1)
            j0 = p0 * 8 + ci
            q = pairid[(min(p0, p1), max(p0, p1))] * 8 + int(ci1)
            j1 = p1 * 8 + int(ci1)
            for c in range(4):
                m = r * 4 + c
                if o2 == 0:
                    kw01p[m, j0] += t4[c, 1]
                    kc[m, 0] += t4[c, 0]
                kw10[m, r] = t4[c, 2]
                kw11[m, r] = t4[c, 3]
            pb10[t, j1] = 1.0
            pb11[t, q] = 1.0

    shifts = [(kh - 1) * _W + (kw - 1) for kh in range(_KH) for kw in range(_KW)]
    kwall = np.zeros((288, 360), np.float64)
    kwall[:, 0:72] = kw01p
    kwall[:, 72:144] = kw10
    kwall[:, 144:216] = kw11
    kwall[:, 216:217] = kc
    pball = np.concatenate([pb10, pb11], axis=1)
    return (kwall.astype(np.float32), pball, shifts)


(_KWALL, _PBALL, _SHIFTS) = _build_constants()


def _body(x_ref, tt_ref, kwall_ref, pball_ref, out_ref):
    f32 = jnp.float32
    bf16 = jnp.bfloat16

    x = x_ref[...]
    s = jnp.sign(x)
    pos = jax.lax.broadcasted_iota(jnp.int32, (1, _L), 1)
    oh = pos // _W
    ow = pos % _W
    parts = []
    for kh in range(_KH):
        for kw in range(_KW):
            d = ((kh - 1) * _W + (kw - 1)) % _L
            if d == 0:
                rolled = s
            else:
                rolled = jnp.concatenate([s[:, d:], s[:, :d]], axis=1)
            ok = (oh + kh - 1 >= 0) & (oh + kh - 1 < _H) & \
                 (ow + kw - 1 >= 0) & (ow + kw - 1 < _W)
            parts.append(rolled * ok.astype(f32))

    ttb = tt_ref[...]
    small = jnp.dot(ttb, kwall_ref[...], preferred_element_type=f32)
    lmat1 = small[:, 0:72]
    w10 = small[:, 72:144]
    w11 = small[:, 144:216]
    cvec = small[:, 216:217]
    col = jax.lax.broadcasted_iota(jnp.int32, (_OC, _OC * _R), 1) // _R
    row = jax.lax.broadcasted_iota(jnp.int32, (_OC, _OC * _R), 0)
    maskd = (col == row).astype(f32)
    d1011 = jnp.concatenate(
        [jnp.concatenate([w10] * _OC, axis=1) * maskd,
         jnp.concatenate([w11] * _OC, axis=1) * maskd], axis=0)
    lm2mm = jnp.dot(d1011, pball_ref[...], preferred_element_type=f32)
    lmat = lmat1 + lm2mm[0:_OC, 0:72]
    mmat = lm2mm[_OC:2 * _OC, 72:360]
    lm = jnp.concatenate([lmat, mmat, cvec], axis=1).astype(bf16)

    ones_row = jnp.ones((1, _L), f32)
    for b in range(_B):
        lo = b * _C
        sb = [p[lo:lo + _C] for p in parts]
        pb = [sb[u] * sb[v] for (u, v) in _PAIRS]
        rhs = jnp.concatenate(sb + pb + [ones_row], axis=0).astype(bf16)
        out_ref[b] = jnp.dot(lm, rhs, preferred_element_type=f32)


def kernel(input, truth_tables):
    x = input.reshape(_B * _C, _L)
    ttb = truth_tables.reshape(_OC, _R * 4)
    out = pl.pallas_call(
        _body,
        out_shape=jax.ShapeDtypeStruct((_B, _OC, _L), jnp.float32),
    )(x, ttb, _KWALL, _PBALL)
    return out.reshape(_B, _OC, _H, _W)
